# baseline (speedup 1.0000x reference)
import functools

import jax
import jax.numpy as jnp
from jax import lax
from jax.experimental import pallas as pl
from jax.experimental.pallas import tpu as pltpu
from jax.experimental.pallas import tpu_sc as plsc

TB = 32
NW = 32
CHG = 16
CHS = 8


def _sc_permute_rows(src, idx3, D, scatter):
    B = src.shape[0]
    _, NCH, CH = idx3.shape
    mesh = plsc.VectorSubcoreMesh(core_axis_name="c", subcore_axis_name="s")

    @functools.partial(
        pl.kernel,
        mesh=mesh,
        out_type=jax.ShapeDtypeStruct((B, D), jnp.float32),
        scratch_types=[
            pltpu.VMEM((NCH, CH), jnp.int32),
            pltpu.VMEM((CH, D), jnp.float32),
            pltpu.VMEM((CH, D), jnp.float32),
            pltpu.SemaphoreType.DMA,
            pltpu.SemaphoreType.DMA,
            pltpu.SemaphoreType.DMA,
            pltpu.SemaphoreType.DMA,
        ],
    )
    def k(src_hbm, idx_hbm, dst_hbm, idx_v, buf0, buf1, r0, r1, w0, w1):
        wid = lax.axis_index("s") * 2 + lax.axis_index("c")
        base = wid * (NCH * CH)
        pltpu.sync_copy(idx_hbm.at[wid], idx_v)
        bufs = (buf0, buf1)
        rsems = (r0, r1)
        wsems = (w0, w1)

        def read_src(j, b):
            if scatter:
                return pltpu.async_copy(
                    src_hbm.at[pl.ds(base + j * CH, CH)], bufs[b], rsems[b]
                )
            return pltpu.async_copy(src_hbm.at[idx_v.at[j]], bufs[b], rsems[b])

        def write_dst(j, b):
            if scatter:
                return pltpu.async_copy(
                    bufs[b], dst_hbm.at[idx_v.at[j]], wsems[b]
                )
            return pltpu.async_copy(
                bufs[b], dst_hbm.at[pl.ds(base + j * CH, CH)], wsems[b]
            )

        hr = [None] * NCH
        hw = [None] * NCH
        hr[0] = read_src(0, 0)
        for j in range(NCH):
            b = j % 2
            hr[j].wait()
            hw[j] = write_dst(j, b)
            if j + 1 < NCH:
                if j - 1 >= 0:
                    hw[j - 1].wait()
                hr[j + 1] = read_src(j + 1, (j + 1) % 2)
        if NCH >= 2:
            hw[NCH - 2].wait()
        hw[NCH - 1].wait()

    return k


def _tc_encode_kernel(starts_ref, x_ref, Ws1_ref, bs1_ref, Ws2_ref, bs2_ref,
                      out_ref, E, MAX_F):
    i = pl.program_id(0)
    t0 = i * TB
    rows = TB * MAX_F
    xb = x_ref[...]
    P = out_ref.shape[-1]
    rowid = lax.broadcasted_iota(jnp.int32, (rows, P), 0)
    for e in range(E):
        lo = jnp.clip(starts_ref[e] - t0, 0, TB)
        hi = jnp.clip(starts_ref[e + 1] - t0, 0, TB)

        @pl.when(hi > lo)
        def _():
            h = jnp.dot(xb, Ws1_ref[e], preferred_element_type=jnp.float32)
            h = h + bs1_ref[e][None, :]
            a = h / (1.0 + jnp.abs(h))
            o = jnp.dot(a, Ws2_ref[e], preferred_element_type=jnp.float32)
            o = o + bs2_ref[e][None, :]
            full = jnp.logical_and(lo == 0, hi == TB)

            @pl.when(full)
            def _():
                out_ref[...] = o

            @pl.when(jnp.logical_not(full))
            def _():
                mask = jnp.logical_and(rowid >= lo * MAX_F, rowid < hi * MAX_F)
                out_ref[...] = jnp.where(mask, o, out_ref[...])


@jax.jit
def kernel(x, Ws1, bs1, Ws2, bs2, eid):
    B, MAX_F, N = x.shape
    E, _, H = Ws1.shape
    P = Ws2.shape[-1]

    oh = (eid[:, None] == jnp.arange(E, dtype=eid.dtype)).astype(jnp.int32)
    counts = jnp.sum(oh, axis=0)
    starts = jnp.concatenate(
        [jnp.zeros((1,), jnp.int32), jnp.cumsum(counts).astype(jnp.int32)]
    )
    rank = jnp.sum(jnp.cumsum(oh, axis=0) * oh, axis=1) - 1
    pos = jnp.sum(oh * starts[None, :E], axis=1) + rank
    starts = jnp.arange(E + 1, dtype=jnp.int32) * (B // E)
    pos = jnp.arange(B, dtype=jnp.int32)

    x2 = x.reshape(B, MAX_F * N)
    posg = pos.reshape(NW, B // (NW * CHG), CHG)
    xs = x2

    grid = B // TB
    grid_spec = pltpu.PrefetchScalarGridSpec(
        num_scalar_prefetch=1,
        grid=(grid,),
        in_specs=[
            pl.BlockSpec((TB * MAX_F, N), lambda i, s: (i, 0)),
            pl.BlockSpec((E, N, H), lambda i, s: (0, 0, 0)),
            pl.BlockSpec((E, H), lambda i, s: (0, 0)),
            pl.BlockSpec((E, H, P), lambda i, s: (0, 0, 0)),
            pl.BlockSpec((E, P), lambda i, s: (0, 0)),
        ],
        out_specs=pl.BlockSpec((TB * MAX_F, P), lambda i, s: (i, 0)),
    )
    outs = pl.pallas_call(
        functools.partial(_tc_encode_kernel, E=E, MAX_F=MAX_F),
        grid_spec=grid_spec,
        out_shape=jax.ShapeDtypeStruct((B * MAX_F, P), jnp.float32),
    )(starts, xs.reshape(B * MAX_F, N), Ws1, bs1, Ws2, bs2)

    outs2 = outs.reshape(B, MAX_F * P)
    poss = pos.reshape(NW, B // (NW * CHS), CHS)
    out = outs2
    return out.reshape(B, MAX_F, P)


# speedup vs baseline: 1.0756x; 1.0756x over previous
import functools

import jax
import jax.numpy as jnp
from jax import lax
from jax.experimental import pallas as pl
from jax.experimental.pallas import tpu as pltpu
from jax.experimental.pallas import tpu_sc as plsc

TB = 32
NW = 32
CHG = 16
CHS = 8


def _sc_permute_rows(src, idx3, Bout, D, scatter):
    _, NCH, CH = idx3.shape
    mesh = plsc.VectorSubcoreMesh(core_axis_name="c", subcore_axis_name="s")

    @functools.partial(
        pl.kernel,
        mesh=mesh,
        out_type=jax.ShapeDtypeStruct((Bout, D), jnp.float32),
        scratch_types=[
            pltpu.VMEM((NCH, CH), jnp.int32),
            pltpu.VMEM((CH, D), jnp.float32),
            pltpu.VMEM((CH, D), jnp.float32),
            pltpu.SemaphoreType.DMA,
            pltpu.SemaphoreType.DMA,
            pltpu.SemaphoreType.DMA,
            pltpu.SemaphoreType.DMA,
        ],
    )
    def k(src_hbm, idx_hbm, dst_hbm, idx_v, buf0, buf1, r0, r1, w0, w1):
        wid = lax.axis_index("s") * 2 + lax.axis_index("c")
        base = wid * (NCH * CH)
        pltpu.sync_copy(idx_hbm.at[wid], idx_v)
        bufs = (buf0, buf1)
        rsems = (r0, r1)
        wsems = (w0, w1)

        def read_src(j, b):
            if scatter:
                return pltpu.async_copy(
                    src_hbm.at[pl.ds(base + j * CH, CH)], bufs[b], rsems[b]
                )
            return pltpu.async_copy(src_hbm.at[idx_v.at[j]], bufs[b], rsems[b])

        def write_dst(j, b):
            if scatter:
                return pltpu.async_copy(
                    bufs[b], dst_hbm.at[idx_v.at[j]], wsems[b]
                )
            return pltpu.async_copy(
                bufs[b], dst_hbm.at[pl.ds(base + j * CH, CH)], wsems[b]
            )

        hr = [None] * NCH
        hw = [None] * NCH
        hr[0] = read_src(0, 0)
        for j in range(NCH):
            b = j % 2
            hr[j].wait()
            hw[j] = write_dst(j, b)
            if j + 1 < NCH:
                if j - 1 >= 0:
                    hw[j - 1].wait()
                hr[j + 1] = read_src(j + 1, (j + 1) % 2)
        if NCH >= 2:
            hw[NCH - 2].wait()
        hw[NCH - 1].wait()

    return k


def _tc_encode_kernel(beid_ref, x_ref, Ws1_ref, bs1_ref, Ws2_ref, bs2_ref,
                      out_ref):
    e = beid_ref[pl.program_id(0)]
    h = jnp.dot(x_ref[...], Ws1_ref[e], preferred_element_type=jnp.float32)
    h = h + bs1_ref[pl.ds(e, 1)]
    a = h / (1.0 + jnp.abs(h))
    o = jnp.dot(a, Ws2_ref[e], preferred_element_type=jnp.float32)
    out_ref[...] = o + bs2_ref[pl.ds(e, 1)]


@jax.jit
def kernel(x, Ws1, bs1, Ws2, bs2, eid):
    B, MAX_F, N = x.shape
    E, _, H = Ws1.shape
    P = Ws2.shape[-1]
    Bp = B + E * TB
    Gp = Bp // TB

    oh = (eid[:, None] == jnp.arange(E, dtype=eid.dtype)).astype(jnp.int32)
    counts = jnp.sum(oh, axis=0)
    pcounts = ((counts + TB - 1) // TB) * TB
    pstarts = jnp.concatenate(
        [jnp.zeros((1,), jnp.int32), jnp.cumsum(pcounts).astype(jnp.int32)]
    )
    rank = jnp.sum(jnp.cumsum(oh, axis=0) * oh, axis=1) - 1
    pos = jnp.sum(oh * pstarts[None, :E], axis=1) + rank
    bstart = jnp.arange(Gp, dtype=jnp.int32) * TB
    block_eid = jnp.minimum(
        jnp.sum((bstart[:, None] >= pstarts[None, 1:]).astype(jnp.int32),
                axis=1),
        E - 1,
    )

    x2 = x.reshape(B, MAX_F * N)
    posg = pos.reshape(NW, B // (NW * CHG), CHG)
    xs = _sc_permute_rows(x2, posg, Bp, MAX_F * N, scatter=True)(x2, posg)

    grid_spec = pltpu.PrefetchScalarGridSpec(
        num_scalar_prefetch=1,
        grid=(Gp,),
        in_specs=[
            pl.BlockSpec((TB * MAX_F, N), lambda i, s: (i, 0)),
            pl.BlockSpec((E, N, H), lambda i, s: (0, 0, 0)),
            pl.BlockSpec((E, H), lambda i, s: (0, 0)),
            pl.BlockSpec((E, H, P), lambda i, s: (0, 0, 0)),
            pl.BlockSpec((E, P), lambda i, s: (0, 0)),
        ],
        out_specs=pl.BlockSpec((TB * MAX_F, P), lambda i, s: (i, 0)),
    )
    outs = pl.pallas_call(
        _tc_encode_kernel,
        grid_spec=grid_spec,
        out_shape=jax.ShapeDtypeStruct((Bp * MAX_F, P), jnp.float32),
    )(block_eid, xs.reshape(Bp * MAX_F, N), Ws1, bs1, Ws2, bs2)

    outs2 = outs.reshape(Bp, MAX_F * P)
    poss = pos.reshape(NW, B // (NW * CHS), CHS)
    out = _sc_permute_rows(outs2, poss, B, MAX_F * P, scatter=False)(
        outs2, poss
    )
    return out.reshape(B, MAX_F, P)


# speedup vs baseline: 3.2007x; 2.9757x over previous
import functools

import jax
import jax.numpy as jnp
from jax import lax
from jax.experimental import pallas as pl
from jax.experimental.pallas import tpu as pltpu

FB = 4


def _encode_kernel(x_ref, W1t_ref, W2t_ref, b1L_ref, b2L_ref, mask_ref,
                   out_ref, E):
    for f in range(FB):
        xf = x_ref[f]
        h = None
        for e in range(E):
            he = jnp.dot(W1t_ref[e], xf, preferred_element_type=jnp.float32)
            he = he * mask_ref[pl.ds(e, 1)]
            h = he if h is None else h + he
        h = h + b1L_ref[...]
        a = h / (1.0 + jnp.abs(h))
        o = None
        for e in range(E):
            oe = jnp.dot(W2t_ref[e], a, preferred_element_type=jnp.float32)
            oe = oe * mask_ref[pl.ds(e, 1)]
            o = oe if o is None else o + oe
        out_ref[f] = o + b2L_ref[...]


@jax.jit
def kernel(x, Ws1, bs1, Ws2, bs2, eid):
    B, MAX_F, N = x.shape
    E, _, H = Ws1.shape
    P = Ws2.shape[-1]

    xt = jnp.transpose(x, (1, 2, 0))
    W1t = jnp.transpose(Ws1, (0, 2, 1))
    W2t = jnp.transpose(Ws2, (0, 2, 1))

    onehot = (eid[None, :] == jnp.arange(E, dtype=eid.dtype)[:, None])
    maskf = onehot.astype(jnp.float32)
    b1L = jnp.matmul(bs1.T, maskf)
    b2L = jnp.matmul(bs2.T, maskf)

    grid = MAX_F // FB
    outT = pl.pallas_call(
        functools.partial(_encode_kernel, E=E),
        grid=(grid,),
        in_specs=[
            pl.BlockSpec((FB, N, B), lambda i: (i, 0, 0)),
            pl.BlockSpec((E, H, N), lambda i: (0, 0, 0)),
            pl.BlockSpec((E, P, H), lambda i: (0, 0, 0)),
            pl.BlockSpec((H, B), lambda i: (0, 0)),
            pl.BlockSpec((P, B), lambda i: (0, 0)),
            pl.BlockSpec((E, B), lambda i: (0, 0)),
        ],
        out_specs=pl.BlockSpec((FB, P, B), lambda i: (i, 0, 0)),
        out_shape=jax.ShapeDtypeStruct((MAX_F, P, B), jnp.float32),
    )(xt, W1t, W2t, b1L, b2L, maskf)

    return jnp.transpose(outT, (2, 0, 1))


# speedup vs baseline: 3.2288x; 1.0088x over previous
import functools

import jax
import jax.numpy as jnp
from jax import lax
from jax.experimental import pallas as pl
from jax.experimental.pallas import tpu as pltpu

FB = 4


def _encode_kernel(x_ref, W1t_ref, W2t_ref, b1L_ref, b2L_ref, mask_ref,
                   out_ref, E):
    for f in range(FB):
        xf = x_ref[f].astype(jnp.bfloat16)
        h = None
        for e in range(E):
            he = jnp.dot(W1t_ref[e].astype(jnp.bfloat16), xf,
                         preferred_element_type=jnp.float32)
            he = he * mask_ref[pl.ds(e, 1)]
            h = he if h is None else h + he
        h = h + b1L_ref[...]
        a = (h / (1.0 + jnp.abs(h))).astype(jnp.bfloat16)
        o = None
        for e in range(E):
            oe = jnp.dot(W2t_ref[e].astype(jnp.bfloat16), a,
                         preferred_element_type=jnp.float32)
            oe = oe * mask_ref[pl.ds(e, 1)]
            o = oe if o is None else o + oe
        out_ref[f] = o + b2L_ref[...]


@jax.jit
def kernel(x, Ws1, bs1, Ws2, bs2, eid):
    B, MAX_F, N = x.shape
    E, _, H = Ws1.shape
    P = Ws2.shape[-1]

    xt = jnp.transpose(x, (1, 2, 0))
    W1t = jnp.transpose(Ws1, (0, 2, 1))
    W2t = jnp.transpose(Ws2, (0, 2, 1))

    onehot = (eid[None, :] == jnp.arange(E, dtype=eid.dtype)[:, None])
    maskf = onehot.astype(jnp.float32)
    b1L = jnp.matmul(bs1.T, maskf)
    b2L = jnp.matmul(bs2.T, maskf)

    grid = MAX_F // FB
    outT = pl.pallas_call(
        functools.partial(_encode_kernel, E=E),
        grid=(grid,),
        in_specs=[
            pl.BlockSpec((FB, N, B), lambda i: (i, 0, 0)),
            pl.BlockSpec((E, H, N), lambda i: (0, 0, 0)),
            pl.BlockSpec((E, P, H), lambda i: (0, 0, 0)),
            pl.BlockSpec((H, B), lambda i: (0, 0)),
            pl.BlockSpec((P, B), lambda i: (0, 0)),
            pl.BlockSpec((E, B), lambda i: (0, 0)),
        ],
        out_specs=pl.BlockSpec((FB, P, B), lambda i: (i, 0, 0)),
        out_shape=jax.ShapeDtypeStruct((MAX_F, P, B), jnp.float32),
    )(xt, W1t, W2t, b1L, b2L, maskf)

    return jnp.transpose(outT, (2, 0, 1))


# speedup vs baseline: 6.4034x; 1.9832x over previous
import functools

import jax
import jax.numpy as jnp
from jax import lax
from jax.experimental import pallas as pl
from jax.experimental.pallas import tpu as pltpu

FB = 4


def _encode_kernel(x_ref, W1c_ref, W2c_ref, b1L_ref, b2L_ref, mask_ref,
                   out_ref, E):
    for f in range(FB):
        xf = x_ref[f].astype(jnp.bfloat16)
        xstack = jnp.concatenate(
            [xf * mask_ref[pl.ds(e, 1)] for e in range(E)], axis=0
        )
        h = jnp.dot(W1c_ref[...], xstack,
                    preferred_element_type=jnp.float32) + b1L_ref[...]
        a = (h / (1.0 + jnp.abs(h))).astype(jnp.bfloat16)
        astack = jnp.concatenate(
            [a * mask_ref[pl.ds(e, 1)] for e in range(E)], axis=0
        )
        o = jnp.dot(W2c_ref[...], astack,
                    preferred_element_type=jnp.float32)
        out_ref[f] = o + b2L_ref[...]


@jax.jit
def kernel(x, Ws1, bs1, Ws2, bs2, eid):
    B, MAX_F, N = x.shape
    E, _, H = Ws1.shape
    P = Ws2.shape[-1]

    xt = jnp.transpose(x, (1, 2, 0))
    W1c = (jnp.transpose(Ws1, (2, 0, 1)).reshape(H, E * N)
           .astype(jnp.bfloat16))
    W2c = (jnp.transpose(Ws2, (2, 0, 1)).reshape(P, E * H)
           .astype(jnp.bfloat16))

    onehot = (eid[None, :] == jnp.arange(E, dtype=eid.dtype)[:, None])
    maskf = onehot.astype(jnp.float32)
    maskb = onehot.astype(jnp.bfloat16)
    b1L = jnp.matmul(bs1.T, maskf)
    b2L = jnp.matmul(bs2.T, maskf)

    grid = MAX_F // FB
    outT = pl.pallas_call(
        functools.partial(_encode_kernel, E=E),
        grid=(grid,),
        in_specs=[
            pl.BlockSpec((FB, N, B), lambda i: (i, 0, 0)),
            pl.BlockSpec((H, E * N), lambda i: (0, 0)),
            pl.BlockSpec((P, E * H), lambda i: (0, 0)),
            pl.BlockSpec((H, B), lambda i: (0, 0)),
            pl.BlockSpec((P, B), lambda i: (0, 0)),
            pl.BlockSpec((E, B), lambda i: (0, 0)),
        ],
        out_specs=pl.BlockSpec((FB, P, B), lambda i: (i, 0, 0)),
        out_shape=jax.ShapeDtypeStruct((MAX_F, P, B), jnp.float32),
    )(xt, W1c, W2c, b1L, b2L, maskb)

    return jnp.transpose(outT, (2, 0, 1))
